# baseline (speedup 1.0000x reference)
import functools

import jax
import jax.numpy as jnp
from jax import lax
from jax.experimental import pallas as pl
from jax.experimental.pallas import tpu as pltpu
from jax.experimental.pallas import tpu_sc as plsc

N = 10000
NP = 10112
H = 128
E = 320000
EL = 100000

NC, NS = 2, 16
CH = 128
EPT = E // NS
KB = 16
NO = 10
PT = KB * NO
EPT_PAD = PT * CH
ACC = 10240
RPT = NP // NS

ELP = 102400
GCH = (2 * ELP) // (NC * NS * CH)


def _conv_accumulate(table, init, src, dst, sidx, didx, rows, acc, sg, ss, c, s):
    pltpu.sync_copy(init.at[c, pl.ds(s * RPT, RPT)], acc.at[pl.ds(s * RPT, RPT)])
    plsc.subcore_barrier()
    rows0, rows1 = rows
    sg0, _ = sg

    def outer(k, carry):
        pltpu.sync_copy(src.at[c, s, pl.ds(k * KB, KB)], sidx)
        pltpu.sync_copy(dst.at[c, s, pl.ds(k * KB, KB)], didx)
        gat = [pltpu.async_copy(table.at[sidx.at[0]], rows0, sg0), None]
        sca = [None, None]
        for j in range(KB):
            b = j & 1
            gat[b].wait()
            sca[b] = pltpu.async_copy(rows[b], acc.at[didx.at[j]], ss[b], add=True)
            if j + 1 < KB:
                nb = (j + 1) & 1
                if sca[nb] is not None:
                    sca[nb].wait()
                gat[nb] = pltpu.async_copy(table.at[sidx.at[j + 1]], rows[nb], sg[nb])
        sca[(KB - 1) & 1].wait()
        sca[(KB - 2) & 1].wait()
        return carry

    lax.fori_loop(0, NO, outer, 0, unroll=False)
    plsc.subcore_barrier()


def _conv_out_body(table, init, src, dst, out, sidx, didx, rows0, rows1,
                   acc, sg0, sg1, ss0, ss1):
    c = lax.axis_index("c")
    s = lax.axis_index("s")
    _conv_accumulate(table, init, src, dst, sidx, didx, (rows0, rows1), acc,
                     (sg0, sg1), (ss0, ss1), c, s)
    pltpu.sync_copy(acc.at[pl.ds(s * RPT, RPT)], out.at[c, pl.ds(s * RPT, RPT)])


GC2 = ELP // (NS * CH)
KB2 = 10
NO2 = GC2 // KB2


def _conv_pairs_body(table, init, src, dst, gidx, gout, sidx, didx, gidxv,
                     rows0, rows1, acc, sg0, sg1, ss0, ss1):
    c = lax.axis_index("c")
    s = lax.axis_index("s")
    _conv_accumulate(table, init, src, dst, sidx, didx, (rows0, rows1), acc,
                     (sg0, sg1), (ss0, ss1), c, s)
    pltpu.sync_copy(gidx.at[c, s], gidxv)
    rows = (rows0, rows1)
    sg = (sg0, sg1)
    ss = (ss0, ss1)
    base = (1 - c) * ELP + s * (GC2 * CH)

    def outer(k, carry):
        j0 = k * KB2
        gat = [pltpu.async_copy(acc.at[gidxv.at[j0]], rows0, sg0), None]
        sca = [None, None]
        for j in range(KB2):
            b = j & 1
            gat[b].wait()
            sca[b] = pltpu.async_copy(
                rows[b], gout.at[pl.ds(base + (j0 + j) * CH, CH)], ss[b])
            if j + 1 < KB2:
                nb = (j + 1) & 1
                if sca[nb] is not None:
                    sca[nb].wait()
                gat[nb] = pltpu.async_copy(acc.at[gidxv.at[j0 + j + 1]], rows[nb], sg[nb])
        sca[(KB2 - 1) & 1].wait()
        sca[(KB2 - 2) & 1].wait()
        return carry

    lax.fori_loop(0, NO2, outer, 0, unroll=False)


@functools.cache
def _sc_kernels():
    mesh = plsc.VectorSubcoreMesh(
        core_axis_name="c", subcore_axis_name="s", num_cores=NC, num_subcores=NS)
    common_scratch = [
        pltpu.VMEM((KB, CH), jnp.int32),
        pltpu.VMEM((KB, CH), jnp.int32),
        pltpu.VMEM((CH, H), jnp.float32),
        pltpu.VMEM((CH, H), jnp.float32),
        pltpu.VMEM_SHARED((ACC, H), jnp.float32),
        pltpu.SemaphoreType.DMA,
        pltpu.SemaphoreType.DMA,
        pltpu.SemaphoreType.DMA,
        pltpu.SemaphoreType.DMA,
    ]
    conv_out = pl.kernel(
        _conv_out_body,
        out_type=jax.ShapeDtypeStruct((NC, NP, H), jnp.float32),
        mesh=mesh,
        scratch_types=common_scratch,
    )
    conv_pairs = pl.kernel(
        _conv_pairs_body,
        out_type=jax.ShapeDtypeStruct((2 * ELP, H), jnp.float32),
        mesh=mesh,
        scratch_types=common_scratch[:2]
        + [pltpu.VMEM((GC2, CH), jnp.int32)]
        + common_scratch[2:],
    )
    return conv_out, conv_pairs


def _leaky(x):
    return jnp.where(x >= 0, x, 0.1 * x)


def _transform_body(act, xm_ref, xs_ref, wm_ref, bm_ref, ws_ref, bs_ref,
                    m_ref, s_ref):
    xm = xm_ref[0]
    xs = xs_ref[0]
    if act:
        xm = _leaky(xm)
        xs = _leaky(xs)
    m_ref[0] = jnp.dot(xm, wm_ref[0], preferred_element_type=jnp.float32) + bm_ref[0, 0]
    s_ref[0] = jnp.dot(xs, ws_ref[0], preferred_element_type=jnp.float32) + bs_ref[0, 0]


def _node_transform(X, Wm, bm, Ws, bs, act):
    BR = 2528
    return pl.pallas_call(
        functools.partial(_transform_body, act),
        grid=(2, NP // BR),
        in_specs=[
            pl.BlockSpec((1, BR, H), lambda t, r: (1 - t, r, 0)),
            pl.BlockSpec((1, BR, H), lambda t, r: (t, r, 0)),
            pl.BlockSpec((1, H, H), lambda t, r: (t, 0, 0)),
            pl.BlockSpec((1, 1, H), lambda t, r: (t, 0, 0)),
            pl.BlockSpec((1, H, H), lambda t, r: (t, 0, 0)),
            pl.BlockSpec((1, 1, H), lambda t, r: (t, 0, 0)),
        ],
        out_specs=[
            pl.BlockSpec((1, BR, H), lambda t, r: (t, r, 0)),
            pl.BlockSpec((1, BR, H), lambda t, r: (t, r, 0)),
        ],
        out_shape=[jax.ShapeDtypeStruct((2, NP, H), jnp.float32)] * 2,
    )(X, X, Wm, bm, Ws, bs)


def _mlp_body(zr_ref, zd_ref, w1a_ref, w1b_ref, b1_ref, w2_ref, b2_ref,
              w3_ref, b3_ref, o_ref):
    h1 = jnp.dot(zr_ref[...], w1a_ref[...], preferred_element_type=jnp.float32)
    h1 = h1 + jnp.dot(zd_ref[...], w1b_ref[...], preferred_element_type=jnp.float32)
    h1 = _leaky(h1 + b1_ref[0])
    h2 = _leaky(jnp.dot(h1, w2_ref[...], preferred_element_type=jnp.float32) + b2_ref[0])
    o_ref[...] = jnp.dot(h2, w3_ref[...], preferred_element_type=jnp.float32) + b3_ref[0]


def _decoder_mlp(G, w1a, w1b, b1, w2, b2, w3, b3):
    BR = 2048
    NB = ELP // BR
    return pl.pallas_call(
        _mlp_body,
        grid=(NB,),
        in_specs=[
            pl.BlockSpec((BR, H), lambda r: (r, 0)),
            pl.BlockSpec((BR, H), lambda r: (r + NB, 0)),
            pl.BlockSpec((H, 2 * H), lambda r: (0, 0)),
            pl.BlockSpec((H, 2 * H), lambda r: (0, 0)),
            pl.BlockSpec((1, 2 * H), lambda r: (0, 0)),
            pl.BlockSpec((2 * H, H), lambda r: (0, 0)),
            pl.BlockSpec((1, H), lambda r: (0, 0)),
            pl.BlockSpec((H, H), lambda r: (0, 0)),
            pl.BlockSpec((1, H), lambda r: (0, 0)),
        ],
        out_specs=pl.BlockSpec((BR, H), lambda r: (r, 0)),
        out_shape=jax.ShapeDtypeStruct((ELP, H), jnp.float32),
    )(G, G, w1a, w1b, b1, w2, b2, w3, b3)


def _pad_tiles(a, padval):
    a = a.reshape(NS, EPT)
    a = jnp.pad(a, ((0, 0), (0, EPT_PAD - EPT)), constant_values=padval)
    return a.reshape(NS, PT, CH)


def kernel(x_rna, x_drug, ei_rd, ei_dr, edge_label_index,
           c1_rd_Wm, c1_rd_bm, c1_rd_Ws, c1_rd_bs,
           c1_dr_Wm, c1_dr_bm, c1_dr_Ws, c1_dr_bs,
           c2_rd_Wm, c2_rd_bm, c2_rd_Ws, c2_rd_bs,
           c2_dr_Wm, c2_dr_bm, c2_dr_Ws, c2_dr_bs,
           dec_W1, dec_b1, dec_W2, dec_b2, dec_W3, dec_b3):
    _conv_pair, _conv_pairs = _sc_kernels()
    SRC = jnp.stack([_pad_tiles(ei_rd[0], 0), _pad_tiles(ei_dr[0] + NP, NP)])
    DST = jnp.stack([_pad_tiles(ei_rd[1], ACC - 1), _pad_tiles(ei_dr[1], ACC - 1)])

    pad_n = ((0, NP - N), (0, 0))
    X1 = jnp.stack([jnp.pad(x_drug, pad_n), jnp.pad(x_rna, pad_n)])
    M1, S1 = _node_transform(
        X1,
        jnp.stack([c1_rd_Wm, c1_dr_Wm]), jnp.stack([c1_rd_bm, c1_dr_bm]).reshape(2, 1, H),
        jnp.stack([c1_rd_Ws, c1_dr_Ws]), jnp.stack([c1_rd_bs, c1_dr_bs]).reshape(2, 1, H),
        act=False)
    O1 = _conv_pair(M1.reshape(2 * NP, H), S1, SRC, DST)

    M2, S2 = _node_transform(
        O1,
        jnp.stack([c2_rd_Wm, c2_dr_Wm]), jnp.stack([c2_rd_bm, c2_dr_bm]).reshape(2, 1, H),
        jnp.stack([c2_rd_Ws, c2_dr_Ws]), jnp.stack([c2_rd_bs, c2_dr_bs]).reshape(2, 1, H),
        act=True)
    gi = jnp.stack([
        jnp.pad(edge_label_index[1], (0, ELP - EL)),
        jnp.pad(edge_label_index[0], (0, ELP - EL)),
    ]).reshape(NC, NS, GC2, CH)
    G = _conv_pairs(M2.reshape(2 * NP, H), S2, SRC, DST, gi)

    o = _decoder_mlp(
        G,
        dec_W1[:H], dec_W1[H:], dec_b1.reshape(1, 2 * H),
        dec_W2, dec_b2.reshape(1, H),
        jnp.pad(dec_W3, ((0, 0), (0, H - 1))), jnp.pad(dec_b3, (0, H - 1)).reshape(1, H))
    return o[:EL, 0]


# speedup vs baseline: 1.0005x; 1.0005x over previous
import functools

import jax
import jax.numpy as jnp
from jax import lax
from jax.experimental import pallas as pl
from jax.experimental.pallas import tpu as pltpu
from jax.experimental.pallas import tpu_sc as plsc

N = 10000
NP = 10112
H = 128
E = 320000
EL = 100000

NC, NS = 2, 16
CH = 128
EPT = E // NS
KB = 16
NO = 10
PT = KB * NO
EPT_PAD = PT * CH
ACC = 10240
RPT = NP // NS

ELP = 102400
GCH = (2 * ELP) // (NC * NS * CH)


def _conv_accumulate(table, init, src, dst, sidx, didx, rows, acc, sg, ss, c, s):
    pltpu.sync_copy(init.at[c, pl.ds(s * RPT, RPT)], acc.at[pl.ds(s * RPT, RPT)])
    plsc.subcore_barrier()
    rows0, rows1 = rows
    sg0, _ = sg

    def outer(k, carry):
        pltpu.sync_copy(src.at[c, s, pl.ds(k * KB, KB)], sidx)
        pltpu.sync_copy(dst.at[c, s, pl.ds(k * KB, KB)], didx)
        gat = [pltpu.async_copy(table.at[sidx.at[0]], rows0, sg0), None]
        sca = [None, None]
        for j in range(KB):
            b = j & 1
            gat[b].wait()
            sca[b] = pltpu.async_copy(rows[b], acc.at[didx.at[j]], ss[b], add=True)
            if j + 1 < KB:
                nb = (j + 1) & 1
                if sca[nb] is not None:
                    sca[nb].wait()
                gat[nb] = pltpu.async_copy(table.at[sidx.at[j + 1]], rows[nb], sg[nb])
        sca[(KB - 1) & 1].wait()
        sca[(KB - 2) & 1].wait()
        return carry

    lax.fori_loop(0, NO, outer, 0, unroll=False)
    plsc.subcore_barrier()


def _conv_out_body(table, init, src, dst, out, sidx, didx, rows0, rows1,
                   acc, sg0, sg1, ss0, ss1):
    c = lax.axis_index("c")
    s = lax.axis_index("s")
    _conv_accumulate(table, init, src, dst, sidx, didx, (rows0, rows1), acc,
                     (sg0, sg1), (ss0, ss1), c, s)
    pltpu.sync_copy(acc.at[pl.ds(s * RPT, RPT)], out.at[c, pl.ds(s * RPT, RPT)])


GC2 = ELP // (NS * CH)
KB2 = 10
NO2 = GC2 // KB2


def _conv_pairs_body(table, init, src, dst, gidx, gout, sidx, didx, gidxv,
                     rows0, rows1, acc, sg0, sg1, ss0, ss1):
    c = lax.axis_index("c")
    s = lax.axis_index("s")
    _conv_accumulate(table, init, src, dst, sidx, didx, (rows0, rows1), acc,
                     (sg0, sg1), (ss0, ss1), c, s)
    pltpu.sync_copy(gidx.at[c, s], gidxv)
    rows = (rows0, rows1)
    sg = (sg0, sg1)
    ss = (ss0, ss1)
    base = (1 - c) * ELP + s * (GC2 * CH)

    def outer(k, carry):
        j0 = k * KB2
        gat = [pltpu.async_copy(acc.at[gidxv.at[j0]], rows0, sg0), None]
        sca = [None, None]
        for j in range(KB2):
            b = j & 1
            gat[b].wait()
            sca[b] = pltpu.async_copy(
                rows[b], gout.at[pl.ds(base + (j0 + j) * CH, CH)], ss[b])
            if j + 1 < KB2:
                nb = (j + 1) & 1
                if sca[nb] is not None:
                    sca[nb].wait()
                gat[nb] = pltpu.async_copy(acc.at[gidxv.at[j0 + j + 1]], rows[nb], sg[nb])
        sca[(KB2 - 1) & 1].wait()
        sca[(KB2 - 2) & 1].wait()
        return carry

    lax.fori_loop(0, NO2, outer, 0, unroll=False)


@functools.cache
def _sc_kernels():
    mesh = plsc.VectorSubcoreMesh(
        core_axis_name="c", subcore_axis_name="s", num_cores=NC, num_subcores=NS)
    common_scratch = [
        pltpu.VMEM((KB, CH), jnp.int32),
        pltpu.VMEM((KB, CH), jnp.int32),
        pltpu.VMEM((CH, H), jnp.float32),
        pltpu.VMEM((CH, H), jnp.float32),
        pltpu.VMEM_SHARED((ACC, H), jnp.float32),
        pltpu.SemaphoreType.DMA,
        pltpu.SemaphoreType.DMA,
        pltpu.SemaphoreType.DMA,
        pltpu.SemaphoreType.DMA,
    ]
    conv_out = pl.kernel(
        _conv_out_body,
        out_type=jax.ShapeDtypeStruct((NC, NP, H), jnp.float32),
        mesh=mesh,
        scratch_types=common_scratch,
    )
    conv_pairs = pl.kernel(
        _conv_pairs_body,
        out_type=jax.ShapeDtypeStruct((2 * ELP, H), jnp.float32),
        mesh=mesh,
        scratch_types=common_scratch[:2]
        + [pltpu.VMEM((GC2, CH), jnp.int32)]
        + common_scratch[2:],
    )
    return conv_out, conv_pairs


def _leaky(x):
    return jnp.where(x >= 0, x, 0.1 * x)


def _transform_body(act, xm_ref, xs_ref, wm_ref, bm_ref, ws_ref, bs_ref,
                    m_ref, s_ref):
    xm = xm_ref[0]
    xs = xs_ref[0]
    if act:
        xm = _leaky(xm)
        xs = _leaky(xs)
    m_ref[0] = jnp.dot(xm, wm_ref[0], preferred_element_type=jnp.float32) + bm_ref[0, 0]
    s_ref[0] = jnp.dot(xs, ws_ref[0], preferred_element_type=jnp.float32) + bs_ref[0, 0]


def _node_transform(X, Wm, bm, Ws, bs, act):
    BR = 2528
    return pl.pallas_call(
        functools.partial(_transform_body, act),
        grid=(2, NP // BR),
        in_specs=[
            pl.BlockSpec((1, BR, H), lambda t, r: (1 - t, r, 0)),
            pl.BlockSpec((1, BR, H), lambda t, r: (t, r, 0)),
            pl.BlockSpec((1, H, H), lambda t, r: (t, 0, 0)),
            pl.BlockSpec((1, 1, H), lambda t, r: (t, 0, 0)),
            pl.BlockSpec((1, H, H), lambda t, r: (t, 0, 0)),
            pl.BlockSpec((1, 1, H), lambda t, r: (t, 0, 0)),
        ],
        out_specs=[
            pl.BlockSpec((1, BR, H), lambda t, r: (t, r, 0)),
            pl.BlockSpec((1, BR, H), lambda t, r: (t, r, 0)),
        ],
        out_shape=[jax.ShapeDtypeStruct((2, NP, H), jnp.float32)] * 2,
    )(X, X, Wm, bm, Ws, bs)


def _mlp_body(zr_ref, zd_ref, w1a_ref, w1b_ref, b1_ref, w2_ref, b2_ref,
              w3_ref, b3_ref, o_ref):
    bf = jnp.bfloat16
    h1 = jnp.dot(zr_ref[...].astype(bf), w1a_ref[...].astype(bf),
                 preferred_element_type=jnp.float32)
    h1 = h1 + jnp.dot(zd_ref[...].astype(bf), w1b_ref[...].astype(bf),
                      preferred_element_type=jnp.float32)
    h1 = _leaky(h1 + b1_ref[0]).astype(bf)
    h2 = _leaky(jnp.dot(h1, w2_ref[...].astype(bf),
                        preferred_element_type=jnp.float32) + b2_ref[0]).astype(bf)
    o_ref[...] = jnp.dot(h2, w3_ref[...].astype(bf),
                         preferred_element_type=jnp.float32) + b3_ref[0]


def _decoder_mlp(G, w1a, w1b, b1, w2, b2, w3, b3):
    BR = 2048
    NB = ELP // BR
    return pl.pallas_call(
        _mlp_body,
        grid=(NB,),
        in_specs=[
            pl.BlockSpec((BR, H), lambda r: (r, 0)),
            pl.BlockSpec((BR, H), lambda r: (r + NB, 0)),
            pl.BlockSpec((H, 2 * H), lambda r: (0, 0)),
            pl.BlockSpec((H, 2 * H), lambda r: (0, 0)),
            pl.BlockSpec((1, 2 * H), lambda r: (0, 0)),
            pl.BlockSpec((2 * H, H), lambda r: (0, 0)),
            pl.BlockSpec((1, H), lambda r: (0, 0)),
            pl.BlockSpec((H, H), lambda r: (0, 0)),
            pl.BlockSpec((1, H), lambda r: (0, 0)),
        ],
        out_specs=pl.BlockSpec((BR, H), lambda r: (r, 0)),
        out_shape=jax.ShapeDtypeStruct((ELP, H), jnp.float32),
    )(G, G, w1a, w1b, b1, w2, b2, w3, b3)


def _pad_tiles(a, padval):
    a = a.reshape(NS, EPT)
    a = jnp.pad(a, ((0, 0), (0, EPT_PAD - EPT)), constant_values=padval)
    return a.reshape(NS, PT, CH)


def kernel(x_rna, x_drug, ei_rd, ei_dr, edge_label_index,
           c1_rd_Wm, c1_rd_bm, c1_rd_Ws, c1_rd_bs,
           c1_dr_Wm, c1_dr_bm, c1_dr_Ws, c1_dr_bs,
           c2_rd_Wm, c2_rd_bm, c2_rd_Ws, c2_rd_bs,
           c2_dr_Wm, c2_dr_bm, c2_dr_Ws, c2_dr_bs,
           dec_W1, dec_b1, dec_W2, dec_b2, dec_W3, dec_b3):
    _conv_pair, _conv_pairs = _sc_kernels()
    SRC = jnp.stack([_pad_tiles(ei_rd[0], 0), _pad_tiles(ei_dr[0] + NP, NP)])
    DST = jnp.stack([_pad_tiles(ei_rd[1], ACC - 1), _pad_tiles(ei_dr[1], ACC - 1)])

    pad_n = ((0, NP - N), (0, 0))
    X1 = jnp.stack([jnp.pad(x_drug, pad_n), jnp.pad(x_rna, pad_n)])
    M1, S1 = _node_transform(
        X1,
        jnp.stack([c1_rd_Wm, c1_dr_Wm]), jnp.stack([c1_rd_bm, c1_dr_bm]).reshape(2, 1, H),
        jnp.stack([c1_rd_Ws, c1_dr_Ws]), jnp.stack([c1_rd_bs, c1_dr_bs]).reshape(2, 1, H),
        act=False)
    O1 = _conv_pair(M1.reshape(2 * NP, H), S1, SRC, DST)

    M2, S2 = _node_transform(
        O1,
        jnp.stack([c2_rd_Wm, c2_dr_Wm]), jnp.stack([c2_rd_bm, c2_dr_bm]).reshape(2, 1, H),
        jnp.stack([c2_rd_Ws, c2_dr_Ws]), jnp.stack([c2_rd_bs, c2_dr_bs]).reshape(2, 1, H),
        act=True)
    gi = jnp.stack([
        jnp.pad(edge_label_index[1], (0, ELP - EL)),
        jnp.pad(edge_label_index[0], (0, ELP - EL)),
    ]).reshape(NC, NS, GC2, CH)
    G = _conv_pairs(M2.reshape(2 * NP, H), S2, SRC, DST, gi)

    o = _decoder_mlp(
        G,
        dec_W1[:H], dec_W1[H:], dec_b1.reshape(1, 2 * H),
        dec_W2, dec_b2.reshape(1, H),
        jnp.pad(dec_W3, ((0, 0), (0, H - 1))), jnp.pad(dec_b3, (0, H - 1)).reshape(1, H))
    return o[:EL, 0]


# speedup vs baseline: 1.5085x; 1.5077x over previous
import functools

import jax
import jax.numpy as jnp
from jax import lax
from jax.experimental import pallas as pl
from jax.experimental.pallas import tpu as pltpu
from jax.experimental.pallas import tpu_sc as plsc

N = 10000
NP = 10240
H = 128
E = 320000
EL = 100000

NC, NS = 2, 16
CH = 112
EPT = E // NS
KB = 12
NO = 15
PT = KB * NO
EPT_PAD = PT * CH
ACC = NP
RPT = NP // NS

ELP = 100352
GC2 = ELP // (NS * CH)
KB3 = 14
NO3 = GC2 // KB3


def _stream_pipe(n, fire_gather, fire_scatter):
    gat = {0: fire_gather(0, 0)}
    if n > 1:
        gat[1] = fire_gather(1, 1)
    sca = {}
    for j in range(n):
        gat[j].wait()
        sca[j] = fire_scatter(j, j % 3)
        nx = j + 2
        if nx < n:
            if nx >= 3:
                sca[nx - 3].wait()
            gat[nx] = fire_gather(nx, nx % 3)
    for j in range(max(0, n - 3), n):
        sca[j].wait()


def _conv_accumulate(table, init, src, dst, sidx, didx, rows, acc, sg, ss, c, s):
    pltpu.sync_copy(init.at[c, pl.ds(s * RPT, RPT)], acc.at[pl.ds(s * RPT, RPT)])
    plsc.subcore_barrier()

    def outer(k, carry):
        pltpu.sync_copy(src.at[c, s, k], sidx)
        pltpu.sync_copy(dst.at[c, s, k], didx)
        _stream_pipe(
            KB,
            lambda j, b: pltpu.async_copy(table.at[sidx.at[j]], rows[b], sg[b]),
            lambda j, b: pltpu.async_copy(rows[b], acc.at[didx.at[j]], ss[b],
                                          add=True),
        )
        return carry

    lax.fori_loop(0, NO, outer, 0, unroll=False)
    plsc.subcore_barrier()


def _conv_out_body(table, init, src, dst, out, sidx, didx, rows0, rows1, rows2,
                   acc, sg0, sg1, sg2, ss0, ss1, ss2):
    c = lax.axis_index("c")
    s = lax.axis_index("s")
    _conv_accumulate(table, init, src, dst, sidx, didx, (rows0, rows1, rows2),
                     acc, (sg0, sg1, sg2), (ss0, ss1, ss2), c, s)
    pltpu.sync_copy(acc.at[pl.ds(s * RPT, RPT)], out.at[c, pl.ds(s * RPT, RPT)])


def _conv_pairs_body(table, init, src, dst, gidx, gout, sidx, didx, gidxv,
                     rows0, rows1, rows2, acc, sg0, sg1, sg2, ss0, ss1, ss2):
    c = lax.axis_index("c")
    s = lax.axis_index("s")
    rows = (rows0, rows1, rows2)
    sg = (sg0, sg1, sg2)
    ss = (ss0, ss1, ss2)
    _conv_accumulate(table, init, src, dst, sidx, didx, rows, acc, sg, ss, c, s)
    base = (1 - c) * ELP + s * (GC2 * CH)

    def outer(k, carry):
        pltpu.sync_copy(gidx.at[c, s, k], gidxv)
        j0 = k * KB3
        _stream_pipe(
            KB3,
            lambda j, b: pltpu.async_copy(acc.at[gidxv.at[j]], rows[b], sg[b]),
            lambda j, b: pltpu.async_copy(
                rows[b], gout.at[pl.ds(base + (j0 + j) * CH, CH)], ss[b]),
        )
        return carry

    lax.fori_loop(0, NO3, outer, 0, unroll=False)


@functools.cache
def _sc_kernels():
    mesh = plsc.VectorSubcoreMesh(
        core_axis_name="c", subcore_axis_name="s", num_cores=NC, num_subcores=NS)
    common_scratch = [
        pltpu.VMEM((KB, CH), jnp.int32),
        pltpu.VMEM((KB, CH), jnp.int32),
        pltpu.VMEM((CH, H), jnp.float32),
        pltpu.VMEM((CH, H), jnp.float32),
        pltpu.VMEM((CH, H), jnp.float32),
        pltpu.VMEM_SHARED((ACC, H), jnp.float32),
        pltpu.SemaphoreType.DMA,
        pltpu.SemaphoreType.DMA,
        pltpu.SemaphoreType.DMA,
        pltpu.SemaphoreType.DMA,
        pltpu.SemaphoreType.DMA,
        pltpu.SemaphoreType.DMA,
    ]
    conv_out = pl.kernel(
        _conv_out_body,
        out_type=jax.ShapeDtypeStruct((NC, NP, H), jnp.float32),
        mesh=mesh,
        scratch_types=common_scratch,
    )
    conv_pairs = pl.kernel(
        _conv_pairs_body,
        out_type=jax.ShapeDtypeStruct((2 * ELP, H), jnp.float32),
        mesh=mesh,
        scratch_types=common_scratch[:2]
        + [pltpu.VMEM((KB3, CH), jnp.int32)]
        + common_scratch[2:],
    )
    return conv_out, conv_pairs


def _leaky(x):
    return jnp.where(x >= 0, x, 0.1 * x)


def _transform_body(act, xm_ref, xs_ref, wm_ref, bm_ref, ws_ref, bs_ref,
                    m_ref, s_ref):
    bf = jnp.bfloat16
    xm = xm_ref[0]
    xs = xs_ref[0]
    if act:
        xm = _leaky(xm)
        xs = _leaky(xs)
    m_ref[0] = jnp.dot(xm.astype(bf), wm_ref[0].astype(bf),
                       preferred_element_type=jnp.float32) + bm_ref[0, 0]
    s_ref[0] = jnp.dot(xs.astype(bf), ws_ref[0].astype(bf),
                       preferred_element_type=jnp.float32) + bs_ref[0, 0]


def _node_transform(X, Wm, bm, Ws, bs, act):
    BR = 2560
    return pl.pallas_call(
        functools.partial(_transform_body, act),
        grid=(2, NP // BR),
        in_specs=[
            pl.BlockSpec((1, BR, H), lambda t, r: (1 - t, r, 0)),
            pl.BlockSpec((1, BR, H), lambda t, r: (t, r, 0)),
            pl.BlockSpec((1, H, H), lambda t, r: (t, 0, 0)),
            pl.BlockSpec((1, 1, H), lambda t, r: (t, 0, 0)),
            pl.BlockSpec((1, H, H), lambda t, r: (t, 0, 0)),
            pl.BlockSpec((1, 1, H), lambda t, r: (t, 0, 0)),
        ],
        out_specs=[
            pl.BlockSpec((1, BR, H), lambda t, r: (t, r, 0)),
            pl.BlockSpec((1, BR, H), lambda t, r: (t, r, 0)),
        ],
        out_shape=[jax.ShapeDtypeStruct((2, NP, H), jnp.float32)] * 2,
    )(X, X, Wm, bm, Ws, bs)


def _mlp_body(zr_ref, zd_ref, w1a_ref, w1b_ref, b1_ref, w2_ref, b2_ref,
              w3_ref, b3_ref, o_ref):
    bf = jnp.bfloat16
    h1 = jnp.dot(zr_ref[...].astype(bf), w1a_ref[...].astype(bf),
                 preferred_element_type=jnp.float32)
    h1 = h1 + jnp.dot(zd_ref[...].astype(bf), w1b_ref[...].astype(bf),
                      preferred_element_type=jnp.float32)
    h1 = _leaky(h1 + b1_ref[0]).astype(bf)
    h2 = _leaky(jnp.dot(h1, w2_ref[...].astype(bf),
                        preferred_element_type=jnp.float32) + b2_ref[0]).astype(bf)
    o_ref[...] = (jnp.dot(h2, w3_ref[...].astype(bf),
                          preferred_element_type=jnp.float32) + b3_ref[0]).astype(bf)


def _decoder_mlp(G, w1a, w1b, b1, w2, b2, w3, b3):
    BR = 2048
    NB = ELP // BR
    return pl.pallas_call(
        _mlp_body,
        grid=(NB,),
        in_specs=[
            pl.BlockSpec((BR, H), lambda r: (r, 0)),
            pl.BlockSpec((BR, H), lambda r: (r + NB, 0)),
            pl.BlockSpec((H, 2 * H), lambda r: (0, 0)),
            pl.BlockSpec((H, 2 * H), lambda r: (0, 0)),
            pl.BlockSpec((1, 2 * H), lambda r: (0, 0)),
            pl.BlockSpec((2 * H, H), lambda r: (0, 0)),
            pl.BlockSpec((1, H), lambda r: (0, 0)),
            pl.BlockSpec((H, H), lambda r: (0, 0)),
            pl.BlockSpec((1, H), lambda r: (0, 0)),
        ],
        out_specs=pl.BlockSpec((BR, H), lambda r: (r, 0)),
        out_shape=jax.ShapeDtypeStruct((ELP, H), jnp.bfloat16),
    )(G, G, w1a, w1b, b1, w2, b2, w3, b3)


def _pad_tiles(a, padval):
    a = a.reshape(NS, EPT)
    a = jnp.pad(a, ((0, 0), (0, EPT_PAD - EPT)), constant_values=padval)
    return a.reshape(NS, NO, KB, CH)


def kernel(x_rna, x_drug, ei_rd, ei_dr, edge_label_index,
           c1_rd_Wm, c1_rd_bm, c1_rd_Ws, c1_rd_bs,
           c1_dr_Wm, c1_dr_bm, c1_dr_Ws, c1_dr_bs,
           c2_rd_Wm, c2_rd_bm, c2_rd_Ws, c2_rd_bs,
           c2_dr_Wm, c2_dr_bm, c2_dr_Ws, c2_dr_bs,
           dec_W1, dec_b1, dec_W2, dec_b2, dec_W3, dec_b3):
    _conv_pair, _conv_pairs = _sc_kernels()
    SRC = jnp.stack([_pad_tiles(ei_rd[0], 0), _pad_tiles(ei_dr[0] + NP, NP)])
    DST = jnp.stack([_pad_tiles(ei_rd[1], ACC - 1), _pad_tiles(ei_dr[1], ACC - 1)])

    pad_n = ((0, NP - N), (0, 0))
    X1 = jnp.stack([jnp.pad(x_drug, pad_n), jnp.pad(x_rna, pad_n)])
    M1, S1 = _node_transform(
        X1,
        jnp.stack([c1_rd_Wm, c1_dr_Wm]), jnp.stack([c1_rd_bm, c1_dr_bm]).reshape(2, 1, H),
        jnp.stack([c1_rd_Ws, c1_dr_Ws]), jnp.stack([c1_rd_bs, c1_dr_bs]).reshape(2, 1, H),
        act=False)
    O1 = _conv_pair(M1.reshape(2 * NP, H), S1, SRC, DST)

    M2, S2 = _node_transform(
        O1,
        jnp.stack([c2_rd_Wm, c2_dr_Wm]), jnp.stack([c2_rd_bm, c2_dr_bm]).reshape(2, 1, H),
        jnp.stack([c2_rd_Ws, c2_dr_Ws]), jnp.stack([c2_rd_bs, c2_dr_bs]).reshape(2, 1, H),
        act=True)
    gi = jnp.stack([
        jnp.pad(edge_label_index[1], (0, ELP - EL)),
        jnp.pad(edge_label_index[0], (0, ELP - EL)),
    ]).reshape(NC, NS, NO3, KB3, CH)
    G = _conv_pairs(M2.reshape(2 * NP, H), S2, SRC, DST, gi)

    o = _decoder_mlp(
        G,
        dec_W1[:H], dec_W1[H:], dec_b1.reshape(1, 2 * H),
        dec_W2, dec_b2.reshape(1, H),
        jnp.pad(dec_W3, ((0, 0), (0, H - 1))), jnp.pad(dec_b3, (0, H - 1)).reshape(1, H))
    return o[:EL, 0].astype(jnp.float32)


# speedup vs baseline: 1.5587x; 1.0332x over previous
import functools

import jax
import jax.numpy as jnp
from jax import lax
from jax.experimental import pallas as pl
from jax.experimental.pallas import tpu as pltpu
from jax.experimental.pallas import tpu_sc as plsc

N = 10000
NP = 10240
H = 128
E = 320000
EL = 100000

NC, NS = 2, 16
CH = 112
EPT = E // NS
KB = 15
NO = 12
PT = KB * NO
EPT_PAD = PT * CH
ACC = NP
RPT = NP // NS

ELP = 100352
GC2 = ELP // (NS * CH)
KB3 = 14
NO3 = GC2 // KB3


def _stream_pipe(n, fire_gather, fire_scatter):
    gat = {0: fire_gather(0, 0)}
    if n > 1:
        gat[1] = fire_gather(1, 1)
    sca = {}
    for j in range(n):
        gat[j].wait()
        sca[j] = fire_scatter(j, j % 3)
        nx = j + 2
        if nx < n:
            if nx >= 3:
                sca[nx - 3].wait()
            gat[nx] = fire_gather(nx, nx % 3)
    for j in range(max(0, n - 3), n):
        sca[j].wait()


def _conv_accumulate(table, init, edges, ibuf, rows, acc, sg, ss, c, s):
    pltpu.sync_copy(init.at[c, pl.ds(s * RPT, RPT)], acc.at[pl.ds(s * RPT, RPT)])
    plsc.subcore_barrier()

    def outer(k, carry):
        pltpu.sync_copy(edges.at[c, s, k], ibuf)
        _stream_pipe(
            KB,
            lambda j, b: pltpu.async_copy(table.at[ibuf.at[0, j]], rows[b], sg[b]),
            lambda j, b: pltpu.async_copy(rows[b], acc.at[ibuf.at[1, j]], ss[b],
                                          add=True),
        )
        return carry

    lax.fori_loop(0, NO, outer, 0, unroll=False)
    plsc.subcore_barrier()


def _conv_out_body(table, init, edges, out, ibuf, rows0, rows1, rows2,
                   acc, sg0, sg1, sg2, ss0, ss1, ss2):
    c = lax.axis_index("c")
    s = lax.axis_index("s")
    _conv_accumulate(table, init, edges, ibuf, (rows0, rows1, rows2),
                     acc, (sg0, sg1, sg2), (ss0, ss1, ss2), c, s)
    pltpu.sync_copy(acc.at[pl.ds(s * RPT, RPT)], out.at[c, pl.ds(s * RPT, RPT)])


def _conv_pairs_body(table, init, edges, gidx, gout, ibuf,
                     rows0, rows1, rows2, acc, sg0, sg1, sg2, ss0, ss1, ss2):
    c = lax.axis_index("c")
    s = lax.axis_index("s")
    rows = (rows0, rows1, rows2)
    sg = (sg0, sg1, sg2)
    ss = (ss0, ss1, ss2)
    _conv_accumulate(table, init, edges, ibuf, rows, acc, sg, ss, c, s)
    base = (1 - c) * ELP + s * (GC2 * CH)

    def outer(k, carry):
        pltpu.sync_copy(gidx.at[c, s, k], ibuf.at[0, pl.ds(0, KB3)])
        j0 = k * KB3
        _stream_pipe(
            KB3,
            lambda j, b: pltpu.async_copy(acc.at[ibuf.at[0, j]], rows[b], sg[b]),
            lambda j, b: pltpu.async_copy(
                rows[b], gout.at[pl.ds(base + (j0 + j) * CH, CH)], ss[b]),
        )
        return carry

    lax.fori_loop(0, NO3, outer, 0, unroll=False)


@functools.cache
def _sc_kernels():
    mesh = plsc.VectorSubcoreMesh(
        core_axis_name="c", subcore_axis_name="s", num_cores=NC, num_subcores=NS)
    common_scratch = [
        pltpu.VMEM((2, KB, CH), jnp.int32),
        pltpu.VMEM((CH, H), jnp.float32),
        pltpu.VMEM((CH, H), jnp.float32),
        pltpu.VMEM((CH, H), jnp.float32),
        pltpu.VMEM_SHARED((ACC, H), jnp.float32),
        pltpu.SemaphoreType.DMA,
        pltpu.SemaphoreType.DMA,
        pltpu.SemaphoreType.DMA,
        pltpu.SemaphoreType.DMA,
        pltpu.SemaphoreType.DMA,
        pltpu.SemaphoreType.DMA,
    ]
    conv_out = pl.kernel(
        _conv_out_body,
        out_type=jax.ShapeDtypeStruct((NC, NP, H), jnp.float32),
        mesh=mesh,
        scratch_types=common_scratch,
    )
    conv_pairs = pl.kernel(
        _conv_pairs_body,
        out_type=jax.ShapeDtypeStruct((2 * ELP, H), jnp.float32),
        mesh=mesh,
        scratch_types=common_scratch,
    )
    return conv_out, conv_pairs


def _leaky(x):
    return jnp.where(x >= 0, x, 0.1 * x)


def _transform_body(act, xm_ref, xs_ref, wm_ref, bm_ref, ws_ref, bs_ref,
                    m_ref, s_ref):
    bf = jnp.bfloat16
    xm = xm_ref[0]
    xs = xs_ref[0]
    if act:
        xm = _leaky(xm)
        xs = _leaky(xs)
    m_ref[0] = jnp.dot(xm.astype(bf), wm_ref[0].astype(bf),
                       preferred_element_type=jnp.float32) + bm_ref[0, 0]
    s_ref[0] = jnp.dot(xs.astype(bf), ws_ref[0].astype(bf),
                       preferred_element_type=jnp.float32) + bs_ref[0, 0]


def _node_transform(X, Wm, bm, Ws, bs, act):
    BR = 2560
    return pl.pallas_call(
        functools.partial(_transform_body, act),
        grid=(2, NP // BR),
        in_specs=[
            pl.BlockSpec((1, BR, H), lambda t, r: (1 - t, r, 0)),
            pl.BlockSpec((1, BR, H), lambda t, r: (t, r, 0)),
            pl.BlockSpec((1, H, H), lambda t, r: (t, 0, 0)),
            pl.BlockSpec((1, 1, H), lambda t, r: (t, 0, 0)),
            pl.BlockSpec((1, H, H), lambda t, r: (t, 0, 0)),
            pl.BlockSpec((1, 1, H), lambda t, r: (t, 0, 0)),
        ],
        out_specs=[
            pl.BlockSpec((1, BR, H), lambda t, r: (t, r, 0)),
            pl.BlockSpec((1, BR, H), lambda t, r: (t, r, 0)),
        ],
        out_shape=[jax.ShapeDtypeStruct((2, NP, H), jnp.float32)] * 2,
    )(X, X, Wm, bm, Ws, bs)


def _mlp_body(zr_ref, zd_ref, w1a_ref, w1b_ref, b1_ref, w2_ref, b2_ref,
              w3_ref, b3_ref, o_ref):
    bf = jnp.bfloat16
    h1 = jnp.dot(zr_ref[...].astype(bf), w1a_ref[...].astype(bf),
                 preferred_element_type=jnp.float32)
    h1 = h1 + jnp.dot(zd_ref[...].astype(bf), w1b_ref[...].astype(bf),
                      preferred_element_type=jnp.float32)
    h1 = _leaky(h1 + b1_ref[0]).astype(bf)
    h2 = _leaky(jnp.dot(h1, w2_ref[...].astype(bf),
                        preferred_element_type=jnp.float32) + b2_ref[0]).astype(bf)
    o_ref[...] = (jnp.dot(h2, w3_ref[...].astype(bf),
                          preferred_element_type=jnp.float32) + b3_ref[0]).astype(bf)


def _decoder_mlp(G, w1a, w1b, b1, w2, b2, w3, b3):
    BR = 2048
    NB = ELP // BR
    return pl.pallas_call(
        _mlp_body,
        grid=(NB,),
        in_specs=[
            pl.BlockSpec((BR, H), lambda r: (r, 0)),
            pl.BlockSpec((BR, H), lambda r: (r + NB, 0)),
            pl.BlockSpec((H, 2 * H), lambda r: (0, 0)),
            pl.BlockSpec((H, 2 * H), lambda r: (0, 0)),
            pl.BlockSpec((1, 2 * H), lambda r: (0, 0)),
            pl.BlockSpec((2 * H, H), lambda r: (0, 0)),
            pl.BlockSpec((1, H), lambda r: (0, 0)),
            pl.BlockSpec((H, H), lambda r: (0, 0)),
            pl.BlockSpec((1, H), lambda r: (0, 0)),
        ],
        out_specs=pl.BlockSpec((BR, H), lambda r: (r, 0)),
        out_shape=jax.ShapeDtypeStruct((ELP, H), jnp.bfloat16),
    )(G, G, w1a, w1b, b1, w2, b2, w3, b3)


def _pad_tiles(a, padval):
    a = a.reshape(NS, EPT)
    a = jnp.pad(a, ((0, 0), (0, EPT_PAD - EPT)), constant_values=padval)
    return a.reshape(NS, NO, 1, KB, CH)


def kernel(x_rna, x_drug, ei_rd, ei_dr, edge_label_index,
           c1_rd_Wm, c1_rd_bm, c1_rd_Ws, c1_rd_bs,
           c1_dr_Wm, c1_dr_bm, c1_dr_Ws, c1_dr_bs,
           c2_rd_Wm, c2_rd_bm, c2_rd_Ws, c2_rd_bs,
           c2_dr_Wm, c2_dr_bm, c2_dr_Ws, c2_dr_bs,
           dec_W1, dec_b1, dec_W2, dec_b2, dec_W3, dec_b3):
    _conv_pair, _conv_pairs = _sc_kernels()
    EDGES = jnp.stack([
        jnp.concatenate([_pad_tiles(ei_rd[0], 0),
                         _pad_tiles(ei_rd[1], ACC - 1)], axis=2),
        jnp.concatenate([_pad_tiles(ei_dr[0] + NP, NP),
                         _pad_tiles(ei_dr[1], ACC - 1)], axis=2),
    ])

    pad_n = ((0, NP - N), (0, 0))
    X1 = jnp.stack([jnp.pad(x_drug, pad_n), jnp.pad(x_rna, pad_n)])
    M1, S1 = _node_transform(
        X1,
        jnp.stack([c1_rd_Wm, c1_dr_Wm]), jnp.stack([c1_rd_bm, c1_dr_bm]).reshape(2, 1, H),
        jnp.stack([c1_rd_Ws, c1_dr_Ws]), jnp.stack([c1_rd_bs, c1_dr_bs]).reshape(2, 1, H),
        act=False)
    O1 = _conv_pair(M1.reshape(2 * NP, H), S1, EDGES)

    M2, S2 = _node_transform(
        O1,
        jnp.stack([c2_rd_Wm, c2_dr_Wm]), jnp.stack([c2_rd_bm, c2_dr_bm]).reshape(2, 1, H),
        jnp.stack([c2_rd_Ws, c2_dr_Ws]), jnp.stack([c2_rd_bs, c2_dr_bs]).reshape(2, 1, H),
        act=True)
    gi = jnp.stack([
        jnp.pad(edge_label_index[1], (0, ELP - EL)),
        jnp.pad(edge_label_index[0], (0, ELP - EL)),
    ]).reshape(NC, NS, NO3, KB3, CH)
    G = _conv_pairs(M2.reshape(2 * NP, H), S2, EDGES, gi)

    o = _decoder_mlp(
        G,
        dec_W1[:H], dec_W1[H:], dec_b1.reshape(1, 2 * H),
        dec_W2, dec_b2.reshape(1, H),
        jnp.pad(dec_W3, ((0, 0), (0, H - 1))), jnp.pad(dec_b3, (0, H - 1)).reshape(1, H))
    return o[:EL, 0].astype(jnp.float32)


# speedup vs baseline: 1.5791x; 1.0131x over previous
import functools

import jax
import jax.numpy as jnp
from jax import lax
from jax.experimental import pallas as pl
from jax.experimental.pallas import tpu as pltpu
from jax.experimental.pallas import tpu_sc as plsc

N = 10000
NP = 10240
H = 128
E = 320000
EL = 100000

NC, NS = 2, 16
CH = 112
EPT = E // NS
KB = 15
NO = 12
PT = KB * NO
EPT_PAD = PT * CH
ACC = NP
RPT = NP // NS

ELP = 100352
GC2 = ELP // (NS * CH)
KB3 = 14
NO3 = GC2 // KB3


def _stream_pipe(n, fire_gather, fire_scatter):
    gat = {0: fire_gather(0, 0)}
    if n > 1:
        gat[1] = fire_gather(1, 1)
    sca = {}
    for j in range(n):
        gat[j].wait()
        sca[j] = fire_scatter(j, j % 3)
        nx = j + 2
        if nx < n:
            if nx >= 3:
                sca[nx - 3].wait()
            gat[nx] = fire_gather(nx, nx % 3)
    for j in range(max(0, n - 3), n):
        sca[j].wait()


def _conv_accumulate(table, init, edges, ibuf, rows, acc, sg, ss, c, s):
    pltpu.sync_copy(init.at[c, pl.ds(s * RPT, RPT)], acc.at[pl.ds(s * RPT, RPT)])
    plsc.subcore_barrier()

    def outer(k, carry):
        pltpu.sync_copy(edges.at[c, s, k], ibuf)
        _stream_pipe(
            KB,
            lambda j, b: pltpu.async_copy(table.at[ibuf.at[0, j]], rows[b], sg[b]),
            lambda j, b: pltpu.async_copy(rows[b], acc.at[ibuf.at[1, j]], ss[b],
                                          add=True),
        )
        return carry

    lax.fori_loop(0, NO, outer, 0, unroll=False)
    plsc.subcore_barrier()


def _conv_out_body(table, init, edges, out, ibuf, rows0, rows1, rows2,
                   acc, sg0, sg1, sg2, ss0, ss1, ss2):
    c = lax.axis_index("c")
    s = lax.axis_index("s")
    _conv_accumulate(table, init, edges, ibuf, (rows0, rows1, rows2),
                     acc, (sg0, sg1, sg2), (ss0, ss1, ss2), c, s)
    pltpu.sync_copy(acc.at[pl.ds(s * RPT, RPT)], out.at[c, pl.ds(s * RPT, RPT)])


def _conv_pairs_body(table, init, edges, gidx, gout, ibuf,
                     rows0, rows1, rows2, acc, sg0, sg1, sg2, ss0, ss1, ss2):
    c = lax.axis_index("c")
    s = lax.axis_index("s")
    rows = (rows0, rows1, rows2)
    sg = (sg0, sg1, sg2)
    ss = (ss0, ss1, ss2)
    _conv_accumulate(table, init, edges, ibuf, rows, acc, sg, ss, c, s)
    base = (1 - c) * ELP + s * (GC2 * CH)

    def outer(k, carry):
        pltpu.sync_copy(gidx.at[c, s, k], ibuf.at[0, pl.ds(0, KB3)])
        j0 = k * KB3
        _stream_pipe(
            KB3,
            lambda j, b: pltpu.async_copy(acc.at[ibuf.at[0, j]], rows[b], sg[b]),
            lambda j, b: pltpu.async_copy(
                rows[b], gout.at[pl.ds(base + (j0 + j) * CH, CH)], ss[b]),
        )
        return carry

    lax.fori_loop(0, NO3, outer, 0, unroll=False)


@functools.cache
def _sc_kernels():
    mesh = plsc.VectorSubcoreMesh(
        core_axis_name="c", subcore_axis_name="s", num_cores=NC, num_subcores=NS)
    common_scratch = [
        pltpu.VMEM((2, KB, CH), jnp.int32),
        pltpu.VMEM((CH, H), jnp.float32),
        pltpu.VMEM((CH, H), jnp.float32),
        pltpu.VMEM((CH, H), jnp.float32),
        pltpu.VMEM_SHARED((ACC, H), jnp.float32),
        pltpu.SemaphoreType.DMA,
        pltpu.SemaphoreType.DMA,
        pltpu.SemaphoreType.DMA,
        pltpu.SemaphoreType.DMA,
        pltpu.SemaphoreType.DMA,
        pltpu.SemaphoreType.DMA,
    ]
    conv_out = pl.kernel(
        _conv_out_body,
        out_type=jax.ShapeDtypeStruct((NC, NP, H), jnp.float32),
        mesh=mesh,
        scratch_types=common_scratch,
    )
    conv_pairs = pl.kernel(
        _conv_pairs_body,
        out_type=jax.ShapeDtypeStruct((2 * ELP, H), jnp.float32),
        mesh=mesh,
        scratch_types=common_scratch,
    )
    return conv_out, conv_pairs


def _leaky(x):
    return jnp.where(x >= 0, x, 0.1 * x)


def _transform_body(act, xm_ref, xs_ref, wm_ref, bm_ref, ws_ref, bs_ref,
                    m_ref, s_ref):
    bf = jnp.bfloat16
    xm = xm_ref[0]
    xs = xs_ref[0]
    if act:
        xm = _leaky(xm)
        xs = _leaky(xs)
    m_ref[0] = jnp.dot(xm.astype(bf), wm_ref[0].astype(bf),
                       preferred_element_type=jnp.float32) + bm_ref[0, 0]
    s_ref[0] = jnp.dot(xs.astype(bf), ws_ref[0].astype(bf),
                       preferred_element_type=jnp.float32) + bs_ref[0, 0]


def _node_transform(X, Wm, bm, Ws, bs, act):
    BR = 2560
    return pl.pallas_call(
        functools.partial(_transform_body, act),
        grid=(2, NP // BR),
        in_specs=[
            pl.BlockSpec((1, BR, H), lambda t, r: (1 - t, r, 0)),
            pl.BlockSpec((1, BR, H), lambda t, r: (t, r, 0)),
            pl.BlockSpec((1, H, H), lambda t, r: (t, 0, 0)),
            pl.BlockSpec((1, 1, H), lambda t, r: (t, 0, 0)),
            pl.BlockSpec((1, H, H), lambda t, r: (t, 0, 0)),
            pl.BlockSpec((1, 1, H), lambda t, r: (t, 0, 0)),
        ],
        out_specs=[
            pl.BlockSpec((1, BR, H), lambda t, r: (t, r, 0)),
            pl.BlockSpec((1, BR, H), lambda t, r: (t, r, 0)),
        ],
        out_shape=[jax.ShapeDtypeStruct((2, NP, H), jnp.float32)] * 2,
    )(X, X, Wm, bm, Ws, bs)


def _mlp_body(zr_ref, zd_ref, w1a_ref, w1b_ref, b1_ref, w2_ref, b2_ref,
              w3_ref, b3_ref, o_ref):
    bf = jnp.bfloat16
    h1 = jnp.dot(zr_ref[...].astype(bf), w1a_ref[...].astype(bf),
                 preferred_element_type=jnp.float32)
    h1 = h1 + jnp.dot(zd_ref[...].astype(bf), w1b_ref[...].astype(bf),
                      preferred_element_type=jnp.float32)
    h1 = _leaky(h1 + b1_ref[0]).astype(bf)
    h2 = _leaky(jnp.dot(h1, w2_ref[...].astype(bf),
                        preferred_element_type=jnp.float32) + b2_ref[0]).astype(bf)
    o_ref[...] = (jnp.dot(h2, w3_ref[...].astype(bf),
                          preferred_element_type=jnp.float32) + b3_ref[0]).astype(bf)


def _decoder_mlp(G, w1a, w1b, b1, w2, b2, w3, b3):
    BR = 3584
    NB = ELP // BR
    return pl.pallas_call(
        _mlp_body,
        grid=(NB,),
        in_specs=[
            pl.BlockSpec((BR, H), lambda r: (r, 0)),
            pl.BlockSpec((BR, H), lambda r: (r + NB, 0)),
            pl.BlockSpec((H, 2 * H), lambda r: (0, 0)),
            pl.BlockSpec((H, 2 * H), lambda r: (0, 0)),
            pl.BlockSpec((1, 2 * H), lambda r: (0, 0)),
            pl.BlockSpec((2 * H, H), lambda r: (0, 0)),
            pl.BlockSpec((1, H), lambda r: (0, 0)),
            pl.BlockSpec((H, H), lambda r: (0, 0)),
            pl.BlockSpec((1, H), lambda r: (0, 0)),
        ],
        out_specs=pl.BlockSpec((BR, H), lambda r: (r, 0)),
        out_shape=jax.ShapeDtypeStruct((ELP, H), jnp.bfloat16),
    )(G, G, w1a, w1b, b1, w2, b2, w3, b3)


def _pad_tiles(a, padval):
    a = a.reshape(NS, EPT)
    a = jnp.pad(a, ((0, 0), (0, EPT_PAD - EPT)), constant_values=padval)
    return a.reshape(NS, NO, 1, KB, CH)


def kernel(x_rna, x_drug, ei_rd, ei_dr, edge_label_index,
           c1_rd_Wm, c1_rd_bm, c1_rd_Ws, c1_rd_bs,
           c1_dr_Wm, c1_dr_bm, c1_dr_Ws, c1_dr_bs,
           c2_rd_Wm, c2_rd_bm, c2_rd_Ws, c2_rd_bs,
           c2_dr_Wm, c2_dr_bm, c2_dr_Ws, c2_dr_bs,
           dec_W1, dec_b1, dec_W2, dec_b2, dec_W3, dec_b3):
    _conv_pair, _conv_pairs = _sc_kernels()
    EDGES = jnp.stack([
        jnp.concatenate([_pad_tiles(ei_rd[0], 0),
                         _pad_tiles(ei_rd[1], ACC - 1)], axis=2),
        jnp.concatenate([_pad_tiles(ei_dr[0] + NP, NP),
                         _pad_tiles(ei_dr[1], ACC - 1)], axis=2),
    ])

    pad_n = ((0, NP - N), (0, 0))
    X1 = jnp.stack([jnp.pad(x_drug, pad_n), jnp.pad(x_rna, pad_n)])
    M1, S1 = _node_transform(
        X1,
        jnp.stack([c1_rd_Wm, c1_dr_Wm]), jnp.stack([c1_rd_bm, c1_dr_bm]).reshape(2, 1, H),
        jnp.stack([c1_rd_Ws, c1_dr_Ws]), jnp.stack([c1_rd_bs, c1_dr_bs]).reshape(2, 1, H),
        act=False)
    O1 = _conv_pair(M1.reshape(2 * NP, H), S1, EDGES)

    M2, S2 = _node_transform(
        O1,
        jnp.stack([c2_rd_Wm, c2_dr_Wm]), jnp.stack([c2_rd_bm, c2_dr_bm]).reshape(2, 1, H),
        jnp.stack([c2_rd_Ws, c2_dr_Ws]), jnp.stack([c2_rd_bs, c2_dr_bs]).reshape(2, 1, H),
        act=True)
    gi = jnp.stack([
        jnp.pad(edge_label_index[1], (0, ELP - EL)),
        jnp.pad(edge_label_index[0], (0, ELP - EL)),
    ]).reshape(NC, NS, NO3, KB3, CH)
    G = _conv_pairs(M2.reshape(2 * NP, H), S2, EDGES, gi)

    o = _decoder_mlp(
        G,
        dec_W1[:H], dec_W1[H:], dec_b1.reshape(1, 2 * H),
        dec_W2, dec_b2.reshape(1, H),
        jnp.pad(dec_W3, ((0, 0), (0, H - 1))), jnp.pad(dec_b3, (0, H - 1)).reshape(1, H))
    return o[:EL, 0].astype(jnp.float32)


# speedup vs baseline: 1.6124x; 1.0211x over previous
import functools

import jax
import jax.numpy as jnp
from jax import lax
from jax.experimental import pallas as pl
from jax.experimental.pallas import tpu as pltpu
from jax.experimental.pallas import tpu_sc as plsc

N = 10000
NP = 10240
H = 128
E = 320000
EL = 100000

NC, NS = 2, 16
CH = 112
EPT = E // NS
KB = 15
NO = 12
PT = KB * NO
EPT_PAD = PT * CH
ACC = NP
RPT = NP // NS

ELP = 100352
GC2 = ELP // (NS * CH)
KB3 = 14
NO3 = GC2 // KB3


def _stream_pipe(n, fire_gather, fire_scatter):
    gat = {0: fire_gather(0, 0)}
    if n > 1:
        gat[1] = fire_gather(1, 1)
    sca = {}
    for j in range(n):
        gat[j].wait()
        sca[j] = fire_scatter(j, j % 3)
        nx = j + 2
        if nx < n:
            if nx >= 3:
                sca[nx - 3].wait()
            gat[nx] = fire_gather(nx, nx % 3)
    for j in range(max(0, n - 3), n):
        sca[j].wait()


def _conv_accumulate(table, init, edges, ibuf, rows, acc, sg, ss, c, s):
    pltpu.sync_copy(init.at[c, pl.ds(s * RPT, RPT)], acc.at[pl.ds(s * RPT, RPT)])
    plsc.subcore_barrier()

    def outer(k, carry):
        pltpu.sync_copy(edges.at[c, s, k], ibuf)
        _stream_pipe(
            KB,
            lambda j, b: pltpu.async_copy(table.at[ibuf.at[0, j]], rows[b], sg[b]),
            lambda j, b: pltpu.async_copy(rows[b], acc.at[ibuf.at[1, j]], ss[b],
                                          add=True),
        )
        return carry

    lax.fori_loop(0, NO, outer, 0, unroll=False)
    plsc.subcore_barrier()


def _conv_out_body(table, init, edges, out, ibuf, rows0, rows1, rows2,
                   acc, sg0, sg1, sg2, ss0, ss1, ss2):
    c = lax.axis_index("c")
    s = lax.axis_index("s")
    _conv_accumulate(table, init, edges, ibuf, (rows0, rows1, rows2),
                     acc, (sg0, sg1, sg2), (ss0, ss1, ss2), c, s)
    pltpu.sync_copy(acc.at[pl.ds(s * RPT, RPT)], out.at[c, pl.ds(s * RPT, RPT)])


def _conv_pairs_body(table, init, edges, gidx, gout, ibuf,
                     rows0, rows1, rows2, acc, sg0, sg1, sg2, ss0, ss1, ss2):
    c = lax.axis_index("c")
    s = lax.axis_index("s")
    rows = (rows0, rows1, rows2)
    sg = (sg0, sg1, sg2)
    ss = (ss0, ss1, ss2)
    _conv_accumulate(table, init, edges, ibuf, rows, acc, sg, ss, c, s)
    base = s * (GC2 * CH)
    col0 = (1 - c) * H

    def outer(k, carry):
        pltpu.sync_copy(gidx.at[c, s, k], ibuf.at[0, pl.ds(0, KB3)])
        j0 = k * KB3
        _stream_pipe(
            KB3,
            lambda j, b: pltpu.async_copy(acc.at[ibuf.at[0, j]], rows[b], sg[b]),
            lambda j, b: pltpu.async_copy(
                rows[b],
                gout.at[pl.ds(base + (j0 + j) * CH, CH), pl.ds(col0, H)],
                ss[b]),
        )
        return carry

    lax.fori_loop(0, NO3, outer, 0, unroll=False)


@functools.cache
def _sc_kernels():
    mesh = plsc.VectorSubcoreMesh(
        core_axis_name="c", subcore_axis_name="s", num_cores=NC, num_subcores=NS)
    common_scratch = [
        pltpu.VMEM((2, KB, CH), jnp.int32),
        pltpu.VMEM((CH, H), jnp.float32),
        pltpu.VMEM((CH, H), jnp.float32),
        pltpu.VMEM((CH, H), jnp.float32),
        pltpu.VMEM_SHARED((ACC, H), jnp.float32),
        pltpu.SemaphoreType.DMA,
        pltpu.SemaphoreType.DMA,
        pltpu.SemaphoreType.DMA,
        pltpu.SemaphoreType.DMA,
        pltpu.SemaphoreType.DMA,
        pltpu.SemaphoreType.DMA,
    ]
    conv_out = pl.kernel(
        _conv_out_body,
        out_type=jax.ShapeDtypeStruct((NC, NP, H), jnp.float32),
        mesh=mesh,
        scratch_types=common_scratch,
    )
    conv_pairs = pl.kernel(
        _conv_pairs_body,
        out_type=jax.ShapeDtypeStruct((ELP, 2 * H), jnp.float32),
        mesh=mesh,
        scratch_types=common_scratch,
    )
    return conv_out, conv_pairs


def _leaky(x):
    return jnp.where(x >= 0, x, 0.1 * x)


def _transform_body(act, xm_ref, xs_ref, wm_ref, bm_ref, ws_ref, bs_ref,
                    m_ref, s_ref):
    bf = jnp.bfloat16
    xm = xm_ref[0]
    xs = xs_ref[0]
    if act:
        xm = _leaky(xm)
        xs = _leaky(xs)
    m_ref[0] = jnp.dot(xm.astype(bf), wm_ref[0].astype(bf),
                       preferred_element_type=jnp.float32) + bm_ref[0, 0]
    s_ref[0] = jnp.dot(xs.astype(bf), ws_ref[0].astype(bf),
                       preferred_element_type=jnp.float32) + bs_ref[0, 0]


def _node_transform(X, Wm, bm, Ws, bs, act):
    BR = 2560
    return pl.pallas_call(
        functools.partial(_transform_body, act),
        grid=(2, NP // BR),
        in_specs=[
            pl.BlockSpec((1, BR, H), lambda t, r: (1 - t, r, 0)),
            pl.BlockSpec((1, BR, H), lambda t, r: (t, r, 0)),
            pl.BlockSpec((1, H, H), lambda t, r: (t, 0, 0)),
            pl.BlockSpec((1, 1, H), lambda t, r: (t, 0, 0)),
            pl.BlockSpec((1, H, H), lambda t, r: (t, 0, 0)),
            pl.BlockSpec((1, 1, H), lambda t, r: (t, 0, 0)),
        ],
        out_specs=[
            pl.BlockSpec((1, BR, H), lambda t, r: (t, r, 0)),
            pl.BlockSpec((1, BR, H), lambda t, r: (t, r, 0)),
        ],
        out_shape=[jax.ShapeDtypeStruct((2, NP, H), jnp.float32)] * 2,
    )(X, X, Wm, bm, Ws, bs)


def _mlp_body(z_ref, w1_ref, b1_ref, w2_ref, b2_ref,
              w3_ref, b3_ref, o_ref):
    bf = jnp.bfloat16
    h1 = jnp.dot(z_ref[...].astype(bf), w1_ref[...].astype(bf),
                 preferred_element_type=jnp.float32)
    h1 = _leaky(h1 + b1_ref[0]).astype(bf)
    h2 = _leaky(jnp.dot(h1, w2_ref[...].astype(bf),
                        preferred_element_type=jnp.float32) + b2_ref[0]).astype(bf)
    o_ref[...] = (jnp.dot(h2, w3_ref[...].astype(bf),
                          preferred_element_type=jnp.float32) + b3_ref[0]).astype(bf)


def _decoder_mlp(G, w1, b1, w2, b2, w3, b3):
    BR = 3584
    NB = ELP // BR
    return pl.pallas_call(
        _mlp_body,
        grid=(NB,),
        in_specs=[
            pl.BlockSpec((BR, 2 * H), lambda r: (r, 0)),
            pl.BlockSpec((2 * H, 2 * H), lambda r: (0, 0)),
            pl.BlockSpec((1, 2 * H), lambda r: (0, 0)),
            pl.BlockSpec((2 * H, H), lambda r: (0, 0)),
            pl.BlockSpec((1, H), lambda r: (0, 0)),
            pl.BlockSpec((H, H), lambda r: (0, 0)),
            pl.BlockSpec((1, H), lambda r: (0, 0)),
        ],
        out_specs=pl.BlockSpec((BR, H), lambda r: (r, 0)),
        out_shape=jax.ShapeDtypeStruct((ELP, H), jnp.bfloat16),
    )(G, w1, b1, w2, b2, w3, b3)


def _pad_tiles(a, padval):
    a = a.reshape(NS, EPT)
    a = jnp.pad(a, ((0, 0), (0, EPT_PAD - EPT)), constant_values=padval)
    return a.reshape(NS, NO, 1, KB, CH)


def kernel(x_rna, x_drug, ei_rd, ei_dr, edge_label_index,
           c1_rd_Wm, c1_rd_bm, c1_rd_Ws, c1_rd_bs,
           c1_dr_Wm, c1_dr_bm, c1_dr_Ws, c1_dr_bs,
           c2_rd_Wm, c2_rd_bm, c2_rd_Ws, c2_rd_bs,
           c2_dr_Wm, c2_dr_bm, c2_dr_Ws, c2_dr_bs,
           dec_W1, dec_b1, dec_W2, dec_b2, dec_W3, dec_b3):
    _conv_pair, _conv_pairs = _sc_kernels()
    EDGES = jnp.stack([
        jnp.concatenate([_pad_tiles(ei_rd[0], 0),
                         _pad_tiles(ei_rd[1], ACC - 1)], axis=2),
        jnp.concatenate([_pad_tiles(ei_dr[0] + NP, NP),
                         _pad_tiles(ei_dr[1], ACC - 1)], axis=2),
    ])

    pad_n = ((0, NP - N), (0, 0))
    X1 = jnp.stack([jnp.pad(x_drug, pad_n), jnp.pad(x_rna, pad_n)])
    M1, S1 = _node_transform(
        X1,
        jnp.stack([c1_rd_Wm, c1_dr_Wm]), jnp.stack([c1_rd_bm, c1_dr_bm]).reshape(2, 1, H),
        jnp.stack([c1_rd_Ws, c1_dr_Ws]), jnp.stack([c1_rd_bs, c1_dr_bs]).reshape(2, 1, H),
        act=False)
    O1 = _conv_pair(M1.reshape(2 * NP, H), S1, EDGES)

    M2, S2 = _node_transform(
        O1,
        jnp.stack([c2_rd_Wm, c2_dr_Wm]), jnp.stack([c2_rd_bm, c2_dr_bm]).reshape(2, 1, H),
        jnp.stack([c2_rd_Ws, c2_dr_Ws]), jnp.stack([c2_rd_bs, c2_dr_bs]).reshape(2, 1, H),
        act=True)
    gi = jnp.stack([
        jnp.pad(edge_label_index[1], (0, ELP - EL)),
        jnp.pad(edge_label_index[0], (0, ELP - EL)),
    ]).reshape(NC, NS, NO3, KB3, CH)
    G = _conv_pairs(M2.reshape(2 * NP, H), S2, EDGES, gi)

    o = _decoder_mlp(
        G,
        dec_W1, dec_b1.reshape(1, 2 * H),
        dec_W2, dec_b2.reshape(1, H),
        jnp.pad(dec_W3, ((0, 0), (0, H - 1))), jnp.pad(dec_b3, (0, H - 1)).reshape(1, H))
    return o[:EL, 0].astype(jnp.float32)
